# baseline (speedup 1.0000x reference)
import functools

import jax
import jax.numpy as jnp
from jax.experimental import pallas as pl
from jax.experimental.pallas import tpu as pltpu

_BK = 128


def _main_kernel(
    e_ref, a_ref, x_ref, y_ref, wfc_ref, bfc_ref, w1_ref, b1_ref, w2_ref,
    b2_ref, o_ref, xe_ref, s1_scr, *, g, m
):
    i = pl.program_id(0)

    @pl.when(i == 0)
    def _prep():
        y_new = (
            jnp.dot(y_ref[:], wfc_ref[:], preferred_element_type=jnp.float32)
            + bfc_ref[:]
        )
        n_nodes = x_ref.shape[0]
        s1_scr[0:n_nodes, :] = jnp.dot(
            x_ref[:], w1_ref[:], preferred_element_type=jnp.float32
        )
        s1_scr[n_nodes:, :] = jnp.dot(
            y_new, w1_ref[:], preferred_element_type=jnp.float32
        )

    xe = jnp.maximum(
        jnp.dot(e_ref[:], s1_scr[:], preferred_element_type=jnp.float32)
        + b1_ref[:],
        0.0,
    )
    xe_ref[:] = xe

    s2_blk = jnp.dot(xe, w2_ref[:], preferred_element_type=jnp.float32)

    @pl.when(i < g - 1)
    def _acc_full():
        c = jnp.dot(a_ref[:], s2_blk, preferred_element_type=jnp.float32)

        @pl.when(i == 0)
        def _():
            o_ref[:] = c

        @pl.when(i > 0)
        def _():
            o_ref[:] = o_ref[:] + c

    @pl.when(i == g - 1)
    def _acc_tail():
        t = m - (g - 1) * _BK
        c = jnp.dot(
            a_ref[:, 0:t], s2_blk[0:t, :], preferred_element_type=jnp.float32
        )
        total = c if g == 1 else o_ref[:] + c
        o_ref[:] = jax.nn.sigmoid(total + b2_ref[:])


def kernel(Y_embedding, X, E_tilde, A_tilde, W_fc, b_fc, W1, b1, W2, b2):
    m = E_tilde.shape[0]
    n = X.shape[0]
    nfeat = X.shape[1]
    nhid = W1.shape[1]
    nclass = W2.shape[1]
    nhigh = Y_embedding.shape[1]
    l = Y_embedding.shape[0]
    f32 = jnp.float32

    bfc2 = b_fc.reshape(1, nfeat)
    b1_2 = b1.reshape(1, nhid)
    b2_2 = b2.reshape(1, nclass)

    g = pl.cdiv(m, _BK)
    const = lambda i: (0, 0)
    body = functools.partial(_main_kernel, g=g, m=m)

    output, x_embedding = pl.pallas_call(
        body,
        grid=(g,),
        in_specs=[
            pl.BlockSpec((_BK, m), lambda i: (i, 0)),
            pl.BlockSpec((m, _BK), lambda i: (0, i)),
            pl.BlockSpec((n, nfeat), const),
            pl.BlockSpec((l, nhigh), const),
            pl.BlockSpec((nhigh, nfeat), const),
            pl.BlockSpec((1, nfeat), const),
            pl.BlockSpec((nfeat, nhid), const),
            pl.BlockSpec((1, nhid), const),
            pl.BlockSpec((nhid, nclass), const),
            pl.BlockSpec((1, nclass), const),
        ],
        out_specs=[
            pl.BlockSpec((m, nclass), const),
            pl.BlockSpec((_BK, nhid), lambda i: (i, 0)),
        ],
        out_shape=[
            jax.ShapeDtypeStruct((m, nclass), f32),
            jax.ShapeDtypeStruct((m, nhid), f32),
        ],
        scratch_shapes=[
            pltpu.VMEM((m, nhid), f32),
        ],
        compiler_params=pltpu.CompilerParams(
            dimension_semantics=("arbitrary",)
        ),
    )(E_tilde, A_tilde, X, Y_embedding, W_fc, bfc2, W1, b1_2, W2, b2_2)

    return (output, x_embedding)


# speedup vs baseline: 1.0592x; 1.0592x over previous
import functools

import jax
import jax.numpy as jnp
from jax.experimental import pallas as pl
from jax.experimental.pallas import tpu as pltpu

_BK = 256


def _prep_kernel(x_ref, y_ref, wfc_ref, bfc_ref, w1_ref, s1_ref):
    y_new = (
        jnp.dot(y_ref[:], wfc_ref[:], preferred_element_type=jnp.float32)
        + bfc_ref[:]
    )
    n_nodes = x_ref.shape[0]
    s1_ref[0:n_nodes, :] = jnp.dot(
        x_ref[:], w1_ref[:], preferred_element_type=jnp.float32
    )
    s1_ref[n_nodes:, :] = jnp.dot(
        y_new, w1_ref[:], preferred_element_type=jnp.float32
    )


def _main_kernel(
    e_ref, a_ref, s1_ref, b1_ref, w2_ref, b2_ref, o_ref, xe_ref, acc_scr,
    sem, *, g, m
):
    i = pl.program_id(0)

    xe = jnp.maximum(
        jnp.dot(e_ref[:], s1_ref[:], preferred_element_type=jnp.float32)
        + b1_ref[:],
        0.0,
    )
    xe_ref[:] = xe

    s2_blk = jnp.dot(xe, w2_ref[:], preferred_element_type=jnp.float32)

    @pl.when(i < g - 1)
    def _acc_full():
        c = jnp.dot(a_ref[:], s2_blk, preferred_element_type=jnp.float32)

        @pl.when(i == 0)
        def _():
            acc_scr[:] = c

        @pl.when(i > 0)
        def _():
            acc_scr[:] = acc_scr[:] + c

    @pl.when(i == g - 1)
    def _acc_tail():
        t = m - (g - 1) * _BK
        c = jnp.dot(
            a_ref[:, 0:t], s2_blk[0:t, :], preferred_element_type=jnp.float32
        )
        total = c if g == 1 else acc_scr[:] + c
        acc_scr[:] = jax.nn.sigmoid(total + b2_ref[:])
        copy = pltpu.make_async_copy(acc_scr, o_ref, sem)
        copy.start()
        copy.wait()


def kernel(Y_embedding, X, E_tilde, A_tilde, W_fc, b_fc, W1, b1, W2, b2):
    m = E_tilde.shape[0]
    nfeat = X.shape[1]
    nhid = W1.shape[1]
    nclass = W2.shape[1]
    f32 = jnp.float32

    bfc2 = b_fc.reshape(1, nfeat)
    b1_2 = b1.reshape(1, nhid)
    b2_2 = b2.reshape(1, nclass)

    g = pl.cdiv(m, _BK)
    const = lambda i: (0, 0)

    s1 = pl.pallas_call(
        _prep_kernel,
        out_shape=jax.ShapeDtypeStruct((m, nhid), f32),
    )(X, Y_embedding, W_fc, bfc2, W1)

    body = functools.partial(_main_kernel, g=g, m=m)

    output, x_embedding = pl.pallas_call(
        body,
        grid=(g,),
        in_specs=[
            pl.BlockSpec((_BK, m), lambda i: (i, 0)),
            pl.BlockSpec((m, _BK), lambda i: (0, i)),
            pl.BlockSpec((m, nhid), const),
            pl.BlockSpec((1, nhid), const),
            pl.BlockSpec((nhid, nclass), const),
            pl.BlockSpec((1, nclass), const),
        ],
        out_specs=[
            pl.BlockSpec(memory_space=pltpu.MemorySpace.HBM),
            pl.BlockSpec((_BK, nhid), lambda i: (i, 0)),
        ],
        out_shape=[
            jax.ShapeDtypeStruct((m, nclass), f32),
            jax.ShapeDtypeStruct((m, nhid), f32),
        ],
        scratch_shapes=[
            pltpu.VMEM((m, nclass), f32),
            pltpu.SemaphoreType.DMA,
        ],
        compiler_params=pltpu.CompilerParams(
            dimension_semantics=("arbitrary",)
        ),
    )(E_tilde, A_tilde, s1, b1_2, W2, b2_2)

    return (output, x_embedding)


# speedup vs baseline: 1.1084x; 1.0465x over previous
import functools

import jax
import jax.numpy as jnp
from jax.experimental import pallas as pl
from jax.experimental.pallas import tpu as pltpu

_BK = 256


def _main_kernel(
    e_ref, a_ref, x_ref, y_ref, wfc_ref, bfc_ref, w1_ref, b1_ref, w2_ref,
    b2_ref, o_ref, xe_ref, s1_scr, *, g, m
):
    i = pl.program_id(0)

    @pl.when(i == 0)
    def _prep():
        y_new = (
            jnp.dot(y_ref[:], wfc_ref[:], preferred_element_type=jnp.float32)
            + bfc_ref[:]
        )
        n_nodes = x_ref.shape[0]
        s1_scr[0:n_nodes, :] = jnp.dot(
            x_ref[:], w1_ref[:], preferred_element_type=jnp.float32
        )
        s1_scr[n_nodes:, :] = jnp.dot(
            y_new, w1_ref[:], preferred_element_type=jnp.float32
        )

    xe = jnp.maximum(
        jnp.dot(e_ref[:], s1_scr[:], preferred_element_type=jnp.float32)
        + b1_ref[:],
        0.0,
    )
    xe_ref[:] = xe

    s2_blk = jnp.dot(xe, w2_ref[:], preferred_element_type=jnp.float32)

    def contrib(s2, a):
        return jax.lax.dot_general(
            s2, a, (((0,), (1,)), ((), ())),
            preferred_element_type=jnp.float32,
        )

    @pl.when(i < g - 1)
    def _acc_full():
        c = contrib(s2_blk, a_ref[:])

        @pl.when(i == 0)
        def _():
            o_ref[:] = c

        @pl.when(i > 0)
        def _():
            o_ref[:] = o_ref[:] + c

    @pl.when(i == g - 1)
    def _acc_tail():
        t = m - (g - 1) * _BK
        c = contrib(s2_blk[0:t, :], a_ref[:, 0:t])
        total = c if g == 1 else o_ref[:] + c
        o_ref[:] = jax.nn.sigmoid(total + b2_ref[:].T)


def kernel(Y_embedding, X, E_tilde, A_tilde, W_fc, b_fc, W1, b1, W2, b2):
    m = E_tilde.shape[0]
    n = X.shape[0]
    nfeat = X.shape[1]
    nhid = W1.shape[1]
    nclass = W2.shape[1]
    nhigh = Y_embedding.shape[1]
    l = Y_embedding.shape[0]
    f32 = jnp.float32

    bfc2 = b_fc.reshape(1, nfeat)
    b1_2 = b1.reshape(1, nhid)
    b2_2 = b2.reshape(1, nclass)

    g = pl.cdiv(m, _BK)
    const = lambda i: (0, 0)
    body = functools.partial(_main_kernel, g=g, m=m)

    output_t, x_embedding = pl.pallas_call(
        body,
        grid=(g,),
        in_specs=[
            pl.BlockSpec((_BK, m), lambda i: (i, 0)),
            pl.BlockSpec((m, _BK), lambda i: (0, i)),
            pl.BlockSpec((n, nfeat), const),
            pl.BlockSpec((l, nhigh), const),
            pl.BlockSpec((nhigh, nfeat), const),
            pl.BlockSpec((1, nfeat), const),
            pl.BlockSpec((nfeat, nhid), const),
            pl.BlockSpec((1, nhid), const),
            pl.BlockSpec((nhid, nclass), const),
            pl.BlockSpec((1, nclass), const),
        ],
        out_specs=[
            pl.BlockSpec((nclass, m), const),
            pl.BlockSpec((_BK, nhid), lambda i: (i, 0)),
        ],
        out_shape=[
            jax.ShapeDtypeStruct((nclass, m), f32),
            jax.ShapeDtypeStruct((m, nhid), f32),
        ],
        scratch_shapes=[
            pltpu.VMEM((m, nhid), f32),
        ],
        compiler_params=pltpu.CompilerParams(
            dimension_semantics=("arbitrary",)
        ),
    )(E_tilde, A_tilde, X, Y_embedding, W_fc, bfc2, W1, b1_2, W2, b2_2)

    return (output_t.T, x_embedding)
